# baseline (speedup 1.0000x reference)
import functools

import jax
import jax.numpy as jnp
from jax import lax
from jax.experimental import pallas as pl
from jax.experimental.pallas import tpu as pltpu
from jax.experimental.pallas import tpu_sc as plsc

N_TOK = 2048
D_IN = 512
CODE_DIM = 256
K = 8192
NUM_Q = 4
NHALF = 2
TB = N_TOK // NHALF
KT = 1024
NSTEPS = K // KT


def _round_init(r, rm2_ref, rn_ref, min_ref, arg_ref):
    rm2_ref[...] = -2.0 * r
    rn_ref[...] = jnp.sum(r * r, axis=1, keepdims=True)
    min_ref[...] = jnp.full((TB, 1), jnp.inf, jnp.float32)
    arg_ref[...] = jnp.zeros((TB, 1), jnp.float32)


def _tile_update(k, cb, rm2_ref, rn_ref, min_ref, arg_ref):
    cbn = jnp.sum(cb * cb, axis=1)
    s2 = lax.dot_general(rm2_ref[...], cb, (((1,), (1,)), ((), ())),
                         preferred_element_type=jnp.float32)
    d = rn_ref[...] + s2 + cbn[None, :]
    m = jnp.min(d, axis=1, keepdims=True)
    iota = lax.broadcasted_iota(jnp.int32, d.shape, 1).astype(jnp.float32)
    cand = jnp.where(d == m, iota, jnp.float32(K))
    a = jnp.min(cand, axis=1, keepdims=True)
    better = m < min_ref[...]
    off = (k * KT).astype(jnp.float32)
    arg_ref[...] = jnp.where(better, a + off, arg_ref[...])
    min_ref[...] = jnp.where(better, m, min_ref[...])


def _emit_idx(k, q, idxf_ref, idxi_ref, arg_ref):
    @pl.when(k == NSTEPS - 1)
    def _():
        a = arg_ref[...]
        idxf_ref[...] = a
        if idxi_ref is not None:
            idxi_ref[...] = a.astype(jnp.int32) + jnp.int32(q * K)


def _first_body(q, w_ref, we_ref, cb_ref, idxf_ref, idxi_ref, r_ref,
                rm2_ref, rn_ref, min_ref, arg_ref):
    k = pl.program_id(0)

    @pl.when(k == 0)
    def _():
        z = lax.dot_general(w_ref[...], we_ref[...],
                            (((1,), (0,)), ((), ())),
                            preferred_element_type=jnp.float32)
        r_ref[...] = z
        _round_init(z, rm2_ref, rn_ref, min_ref, arg_ref)

    _tile_update(k, cb_ref[0], rm2_ref, rn_ref, min_ref, arg_ref)
    _emit_idx(k, q, idxf_ref, idxi_ref, arg_ref)


def _next_body(q, rp_ref, sel_ref, cb_ref, idxf_ref, idxi_ref, r_ref,
               rm2_ref, rn_ref, min_ref, arg_ref):
    k = pl.program_id(0)

    @pl.when(k == 0)
    def _():
        r = rp_ref[...] - sel_ref[...]
        r_ref[...] = r
        _round_init(r, rm2_ref, rn_ref, min_ref, arg_ref)

    _tile_update(k, cb_ref[0], rm2_ref, rn_ref, min_ref, arg_ref)
    _emit_idx(k, q, idxf_ref, idxi_ref, arg_ref)


def _last_body(q, rp_ref, sel_ref, cb_ref, idxf_ref,
               rm2_ref, rn_ref, min_ref, arg_ref):
    k = pl.program_id(0)

    @pl.when(k == 0)
    def _():
        r = rp_ref[...] - sel_ref[...]
        _round_init(r, rm2_ref, rn_ref, min_ref, arg_ref)

    _tile_update(k, cb_ref[0], rm2_ref, rn_ref, min_ref, arg_ref)
    _emit_idx(k, q, idxf_ref, None, arg_ref)


_SCRATCH = [
    pltpu.VMEM((TB, CODE_DIM), jnp.float32),
    pltpu.VMEM((TB, 1), jnp.float32),
    pltpu.VMEM((TB, 1), jnp.float32),
    pltpu.VMEM((TB, 1), jnp.float32),
]


def _cb_spec(q):
    return pl.BlockSpec((1, KT, CODE_DIM), lambda k: (q, k, 0))


_PARAMS = pltpu.CompilerParams(dimension_semantics=("arbitrary",))


def _tok_spec(cols):
    return pl.BlockSpec((TB, cols), lambda k: (0, 0))


_IDX_OUT = [
    jax.ShapeDtypeStruct((TB, 1), jnp.float32),
    jax.ShapeDtypeStruct((TB, 1), jnp.int32),
]


def _vq_first(h, weights, W_enc, codebooks):
    return pl.pallas_call(
        functools.partial(_first_body, 0),
        grid=(NSTEPS,),
        in_specs=[pl.BlockSpec((TB, D_IN), lambda k, _h=h: (_h, 0)),
                  pl.BlockSpec((D_IN, CODE_DIM), lambda k: (0, 0)),
                  _cb_spec(0)],
        out_specs=[_tok_spec(1), _tok_spec(1), _tok_spec(CODE_DIM)],
        out_shape=_IDX_OUT + [
            jax.ShapeDtypeStruct((TB, CODE_DIM), jnp.float32)],
        scratch_shapes=_SCRATCH,
        compiler_params=_PARAMS,
    )(weights, W_enc, codebooks)


def _vq_next(q, r_prev, sel_prev, codebooks):
    return pl.pallas_call(
        functools.partial(_next_body, q),
        grid=(NSTEPS,),
        in_specs=[_tok_spec(CODE_DIM), _tok_spec(CODE_DIM), _cb_spec(q)],
        out_specs=[_tok_spec(1), _tok_spec(1), _tok_spec(CODE_DIM)],
        out_shape=_IDX_OUT + [
            jax.ShapeDtypeStruct((TB, CODE_DIM), jnp.float32)],
        scratch_shapes=_SCRATCH,
        compiler_params=_PARAMS,
    )(r_prev, sel_prev, codebooks)


def _vq_last(q, r_prev, sel_prev, codebooks):
    return pl.pallas_call(
        functools.partial(_last_body, q),
        grid=(NSTEPS,),
        in_specs=[_tok_spec(CODE_DIM), _tok_spec(CODE_DIM), _cb_spec(q)],
        out_specs=[_tok_spec(1)],
        out_shape=[jax.ShapeDtypeStruct((TB, 1), jnp.float32)],
        scratch_shapes=_SCRATCH,
        compiler_params=_PARAMS,
    )(r_prev, sel_prev, codebooks)


_GATHER_CACHE = {}


def _sc_gather(flat_cb, idx):
    n = idx.shape[0]
    if n not in _GATHER_CACHE:
        info = plsc.get_sparse_core_info()
        nw = info.num_cores * info.num_subcores
        b_per_w = n // nw
        mesh = plsc.VectorSubcoreMesh(core_axis_name="c",
                                      subcore_axis_name="s")

        @functools.partial(
            pl.kernel,
            mesh=mesh,
            out_type=jax.ShapeDtypeStruct((n, CODE_DIM), jnp.float32),
            scratch_types=[
                pltpu.VMEM((b_per_w,), jnp.int32),
                pltpu.VMEM((b_per_w, CODE_DIM), jnp.float32),
                pltpu.SemaphoreType.DMA,
            ],
        )
        def gather_kernel(table_hbm, idx_hbm, out_hbm, idx_v, rows_v, sem):
            wid = lax.axis_index("s") * info.num_cores + lax.axis_index("c")
            base = wid * b_per_w
            pltpu.sync_copy(idx_hbm.at[pl.ds(base, b_per_w)], idx_v)
            pltpu.async_copy(table_hbm.at[idx_v], rows_v, sem).wait()
            pltpu.sync_copy(rows_v, out_hbm.at[pl.ds(base, b_per_w)])

        _GATHER_CACHE[n] = gather_kernel
    return _GATHER_CACHE[n](flat_cb, idx)


def kernel(weights, y, W_enc, codebooks):
    flat_cb = codebooks.reshape(NUM_Q * K, CODE_DIM)
    idx_cols = [[], []]
    r = [None, None]
    idxi = [None, None]
    for h in range(NHALF):
        f, ii, rr = _vq_first(h, weights, W_enc, codebooks)
        idx_cols[h].append(f)
        idxi[h], r[h] = ii, rr
    for q in range(1, NUM_Q):
        for h in range(NHALF):
            sel = _sc_gather(flat_cb, idxi[h].reshape(TB))
            if q < NUM_Q - 1:
                f, ii, rr = _vq_next(q, r[h], sel, codebooks)
                idxi[h], r[h] = ii, rr
            else:
                (f,) = _vq_last(q, r[h], sel, codebooks)
            idx_cols[h].append(f)
    indices = jnp.concatenate(
        [jnp.concatenate(cols, axis=1) for cols in idx_cols], axis=0)
    bos = jnp.array([K], dtype=jnp.float32)
    eos = jnp.array([K + 1], dtype=jnp.float32)
    x = jnp.concatenate([bos, indices.reshape(-1), eos])
    return (x, y)


# speedup vs baseline: 1.1205x; 1.1205x over previous
import functools

import jax
import jax.numpy as jnp
from jax import lax
from jax.experimental import pallas as pl
from jax.experimental.pallas import tpu as pltpu
from jax.experimental.pallas import tpu_sc as plsc

N_TOK = 2048
D_IN = 512
CODE_DIM = 256
K = 8192
NUM_Q = 4
NHALF = 2
TB = N_TOK // NHALF
KT = 1024
NSTEPS = K // KT


NLANE = 128
NSLOT = KT // NLANE


def _round_init(r, rm2_ref, rn_ref, cv_ref, cs_ref):
    rm2_ref[...] = -2.0 * r
    rn_ref[...] = jnp.sum(r * r, axis=1, keepdims=True)
    cv_ref[...] = jnp.full((TB, NLANE), jnp.inf, jnp.float32)
    cs_ref[...] = jnp.zeros((TB, NLANE), jnp.float32)


def _tile_update(k, cb, rm2_ref, rn_ref, cv_ref, cs_ref):
    cbn = jnp.sum(cb * cb, axis=1)
    s2 = lax.dot_general(rm2_ref[...], cb, (((1,), (1,)), ((), ())),
                         preferred_element_type=jnp.float32)
    d = rn_ref[...] + s2 + cbn[None, :]
    cv = cv_ref[...]
    cs = cs_ref[...]
    for v in range(NSLOT):
        dv = lax.slice(d, (0, v * NLANE), (TB, (v + 1) * NLANE))
        slot = (k * NSLOT + v).astype(jnp.float32)
        mask = dv < cv
        cs = jnp.where(mask, slot, cs)
        cv = jnp.minimum(dv, cv)
    cv_ref[...] = cv
    cs_ref[...] = cs


def _emit_idx(k, q, idxf_ref, idxi_ref, cv_ref, cs_ref):
    @pl.when(k == NSTEPS - 1)
    def _():
        cv = cv_ref[...]
        cs = cs_ref[...]
        m = jnp.min(cv, axis=1, keepdims=True)
        lane = lax.broadcasted_iota(jnp.int32, cv.shape, 1).astype(jnp.float32)
        cand = jnp.where(cv == m, cs * jnp.float32(NLANE) + lane,
                         jnp.float32(2 * K))
        a = jnp.min(cand, axis=1, keepdims=True)
        idxf_ref[...] = a
        if idxi_ref is not None:
            idxi_ref[...] = a.astype(jnp.int32) + jnp.int32(q * K)


def _first_body(q, w_ref, we_ref, cb_ref, idxf_ref, idxi_ref, r_ref,
                rm2_ref, rn_ref, cv_ref, cs_ref):
    k = pl.program_id(0)

    @pl.when(k == 0)
    def _():
        z = lax.dot_general(w_ref[...], we_ref[...],
                            (((1,), (0,)), ((), ())),
                            preferred_element_type=jnp.float32)
        r_ref[...] = z
        _round_init(z, rm2_ref, rn_ref, cv_ref, cs_ref)

    _tile_update(k, cb_ref[0], rm2_ref, rn_ref, cv_ref, cs_ref)
    _emit_idx(k, q, idxf_ref, idxi_ref, cv_ref, cs_ref)


def _next_body(q, rp_ref, sel_ref, cb_ref, idxf_ref, idxi_ref, r_ref,
               rm2_ref, rn_ref, cv_ref, cs_ref):
    k = pl.program_id(0)

    @pl.when(k == 0)
    def _():
        r = rp_ref[...] - sel_ref[...]
        r_ref[...] = r
        _round_init(r, rm2_ref, rn_ref, cv_ref, cs_ref)

    _tile_update(k, cb_ref[0], rm2_ref, rn_ref, cv_ref, cs_ref)
    _emit_idx(k, q, idxf_ref, idxi_ref, cv_ref, cs_ref)


def _last_body(q, rp_ref, sel_ref, cb_ref, idxf_ref,
               rm2_ref, rn_ref, cv_ref, cs_ref):
    k = pl.program_id(0)

    @pl.when(k == 0)
    def _():
        r = rp_ref[...] - sel_ref[...]
        _round_init(r, rm2_ref, rn_ref, cv_ref, cs_ref)

    _tile_update(k, cb_ref[0], rm2_ref, rn_ref, cv_ref, cs_ref)
    _emit_idx(k, q, idxf_ref, None, cv_ref, cs_ref)


_SCRATCH = [
    pltpu.VMEM((TB, CODE_DIM), jnp.float32),
    pltpu.VMEM((TB, 1), jnp.float32),
    pltpu.VMEM((TB, NLANE), jnp.float32),
    pltpu.VMEM((TB, NLANE), jnp.float32),
]


def _cb_spec(q):
    return pl.BlockSpec((1, KT, CODE_DIM), lambda k: (q, k, 0))


_PARAMS = pltpu.CompilerParams(dimension_semantics=("arbitrary",))


def _tok_spec(cols):
    return pl.BlockSpec((TB, cols), lambda k: (0, 0))


_IDX_OUT = [
    jax.ShapeDtypeStruct((TB, 1), jnp.float32),
    jax.ShapeDtypeStruct((TB, 1), jnp.int32),
]


def _vq_first(h, weights, W_enc, codebooks):
    return pl.pallas_call(
        functools.partial(_first_body, 0),
        grid=(NSTEPS,),
        in_specs=[pl.BlockSpec((TB, D_IN), lambda k, _h=h: (_h, 0)),
                  pl.BlockSpec((D_IN, CODE_DIM), lambda k: (0, 0)),
                  _cb_spec(0)],
        out_specs=[_tok_spec(1), _tok_spec(1), _tok_spec(CODE_DIM)],
        out_shape=_IDX_OUT + [
            jax.ShapeDtypeStruct((TB, CODE_DIM), jnp.float32)],
        scratch_shapes=_SCRATCH,
        compiler_params=_PARAMS,
    )(weights, W_enc, codebooks)


def _vq_next(q, r_prev, sel_prev, codebooks):
    return pl.pallas_call(
        functools.partial(_next_body, q),
        grid=(NSTEPS,),
        in_specs=[_tok_spec(CODE_DIM), _tok_spec(CODE_DIM), _cb_spec(q)],
        out_specs=[_tok_spec(1), _tok_spec(1), _tok_spec(CODE_DIM)],
        out_shape=_IDX_OUT + [
            jax.ShapeDtypeStruct((TB, CODE_DIM), jnp.float32)],
        scratch_shapes=_SCRATCH,
        compiler_params=_PARAMS,
    )(r_prev, sel_prev, codebooks)


def _vq_last(q, r_prev, sel_prev, codebooks):
    return pl.pallas_call(
        functools.partial(_last_body, q),
        grid=(NSTEPS,),
        in_specs=[_tok_spec(CODE_DIM), _tok_spec(CODE_DIM), _cb_spec(q)],
        out_specs=[_tok_spec(1)],
        out_shape=[jax.ShapeDtypeStruct((TB, 1), jnp.float32)],
        scratch_shapes=_SCRATCH,
        compiler_params=_PARAMS,
    )(r_prev, sel_prev, codebooks)


_GATHER_CACHE = {}


def _sc_gather(flat_cb, idx):
    n = idx.shape[0]
    if n not in _GATHER_CACHE:
        info = plsc.get_sparse_core_info()
        nw = info.num_cores * info.num_subcores
        b_per_w = n // nw
        mesh = plsc.VectorSubcoreMesh(core_axis_name="c",
                                      subcore_axis_name="s")

        @functools.partial(
            pl.kernel,
            mesh=mesh,
            out_type=jax.ShapeDtypeStruct((n, CODE_DIM), jnp.float32),
            scratch_types=[
                pltpu.VMEM((b_per_w,), jnp.int32),
                pltpu.VMEM((b_per_w, CODE_DIM), jnp.float32),
                pltpu.SemaphoreType.DMA,
            ],
        )
        def gather_kernel(table_hbm, idx_hbm, out_hbm, idx_v, rows_v, sem):
            wid = lax.axis_index("s") * info.num_cores + lax.axis_index("c")
            base = wid * b_per_w
            pltpu.sync_copy(idx_hbm.at[pl.ds(base, b_per_w)], idx_v)
            pltpu.async_copy(table_hbm.at[idx_v], rows_v, sem).wait()
            pltpu.sync_copy(rows_v, out_hbm.at[pl.ds(base, b_per_w)])

        _GATHER_CACHE[n] = gather_kernel
    return _GATHER_CACHE[n](flat_cb, idx)


def kernel(weights, y, W_enc, codebooks):
    flat_cb = codebooks.reshape(NUM_Q * K, CODE_DIM)
    idx_cols = [[], []]
    r = [None, None]
    idxi = [None, None]
    for h in range(NHALF):
        f, ii, rr = _vq_first(h, weights, W_enc, codebooks)
        idx_cols[h].append(f)
        idxi[h], r[h] = ii, rr
    for q in range(1, NUM_Q):
        for h in range(NHALF):
            sel = _sc_gather(flat_cb, idxi[h].reshape(TB))
            if q < NUM_Q - 1:
                f, ii, rr = _vq_next(q, r[h], sel, codebooks)
                idxi[h], r[h] = ii, rr
            else:
                (f,) = _vq_last(q, r[h], sel, codebooks)
            idx_cols[h].append(f)
    indices = jnp.concatenate(
        [jnp.concatenate(cols, axis=1) for cols in idx_cols], axis=0)
    bos = jnp.array([K], dtype=jnp.float32)
    eos = jnp.array([K + 1], dtype=jnp.float32)
    x = jnp.concatenate([bos, indices.reshape(-1), eos])
    return (x, y)


# speedup vs baseline: 1.2634x; 1.1275x over previous
import functools

import jax
import jax.numpy as jnp
from jax import lax
from jax.experimental import pallas as pl
from jax.experimental.pallas import tpu as pltpu
from jax.experimental.pallas import tpu_sc as plsc

N_TOK = 2048
D_IN = 512
CODE_DIM = 256
K = 8192
NUM_Q = 4
NHALF = 1
TB = N_TOK // NHALF
KT = 1024
NSTEPS = K // KT


NLANE = 128
NSLOT = KT // NLANE


def _round_init(r, rm2_ref, rn_ref, cv_ref, cs_ref):
    rm2_ref[...] = -2.0 * r
    rn_ref[...] = jnp.sum(r * r, axis=1, keepdims=True)
    cv_ref[...] = jnp.full((TB, NLANE), jnp.inf, jnp.float32)
    cs_ref[...] = jnp.zeros((TB, NLANE), jnp.float32)


def _tile_update(k, cb, rm2_ref, rn_ref, cv_ref, cs_ref):
    cbn = jnp.sum(cb * cb, axis=1)
    s2 = lax.dot_general(rm2_ref[...], cb, (((1,), (1,)), ((), ())),
                         preferred_element_type=jnp.float32)
    d = rn_ref[...] + s2 + cbn[None, :]
    cv = cv_ref[...]
    cs = cs_ref[...]
    for v in range(NSLOT):
        dv = lax.slice(d, (0, v * NLANE), (TB, (v + 1) * NLANE))
        slot = (k * NSLOT + v).astype(jnp.float32)
        mask = dv < cv
        cs = jnp.where(mask, slot, cs)
        cv = jnp.minimum(dv, cv)
    cv_ref[...] = cv
    cs_ref[...] = cs


def _emit_idx(k, q, idxf_ref, idxi_ref, cv_ref, cs_ref):
    @pl.when(k == NSTEPS - 1)
    def _():
        cv = cv_ref[...]
        cs = cs_ref[...]
        m = jnp.min(cv, axis=1, keepdims=True)
        lane = lax.broadcasted_iota(jnp.int32, cv.shape, 1).astype(jnp.float32)
        cand = jnp.where(cv == m, cs * jnp.float32(NLANE) + lane,
                         jnp.float32(2 * K))
        a = jnp.min(cand, axis=1, keepdims=True)
        idxf_ref[...] = a
        if idxi_ref is not None:
            idxi_ref[...] = a.astype(jnp.int32) + jnp.int32(q * K)


def _first_body(q, w_ref, we_ref, cb_ref, idxf_ref, idxi_ref, r_ref,
                rm2_ref, rn_ref, cv_ref, cs_ref):
    k = pl.program_id(0)

    @pl.when(k == 0)
    def _():
        z = lax.dot_general(w_ref[...], we_ref[...],
                            (((1,), (0,)), ((), ())),
                            preferred_element_type=jnp.float32)
        r_ref[...] = z
        _round_init(z, rm2_ref, rn_ref, cv_ref, cs_ref)

    _tile_update(k, cb_ref[0], rm2_ref, rn_ref, cv_ref, cs_ref)
    _emit_idx(k, q, idxf_ref, idxi_ref, cv_ref, cs_ref)


def _next_body(q, rp_ref, sel_ref, cb_ref, idxf_ref, idxi_ref, r_ref,
               rm2_ref, rn_ref, cv_ref, cs_ref):
    k = pl.program_id(0)

    @pl.when(k == 0)
    def _():
        r = rp_ref[...] - sel_ref[...]
        r_ref[...] = r
        _round_init(r, rm2_ref, rn_ref, cv_ref, cs_ref)

    _tile_update(k, cb_ref[0], rm2_ref, rn_ref, cv_ref, cs_ref)
    _emit_idx(k, q, idxf_ref, idxi_ref, cv_ref, cs_ref)


def _last_body(q, rp_ref, sel_ref, cb_ref, idxf_ref,
               rm2_ref, rn_ref, cv_ref, cs_ref):
    k = pl.program_id(0)

    @pl.when(k == 0)
    def _():
        r = rp_ref[...] - sel_ref[...]
        _round_init(r, rm2_ref, rn_ref, cv_ref, cs_ref)

    _tile_update(k, cb_ref[0], rm2_ref, rn_ref, cv_ref, cs_ref)
    _emit_idx(k, q, idxf_ref, None, cv_ref, cs_ref)


_SCRATCH = [
    pltpu.VMEM((TB, CODE_DIM), jnp.float32),
    pltpu.VMEM((TB, 1), jnp.float32),
    pltpu.VMEM((TB, NLANE), jnp.float32),
    pltpu.VMEM((TB, NLANE), jnp.float32),
]


def _cb_spec(q):
    return pl.BlockSpec((1, KT, CODE_DIM), lambda k: (q, k, 0))


_PARAMS = pltpu.CompilerParams(dimension_semantics=("arbitrary",))


def _tok_spec(cols):
    return pl.BlockSpec((TB, cols), lambda k: (0, 0))


_IDX_OUT = [
    jax.ShapeDtypeStruct((TB, 1), jnp.float32),
    jax.ShapeDtypeStruct((TB, 1), jnp.int32),
]


def _vq_first(h, weights, W_enc, codebooks):
    return pl.pallas_call(
        functools.partial(_first_body, 0),
        grid=(NSTEPS,),
        in_specs=[pl.BlockSpec((TB, D_IN), lambda k, _h=h: (_h, 0)),
                  pl.BlockSpec((D_IN, CODE_DIM), lambda k: (0, 0)),
                  _cb_spec(0)],
        out_specs=[_tok_spec(1), _tok_spec(1), _tok_spec(CODE_DIM)],
        out_shape=_IDX_OUT + [
            jax.ShapeDtypeStruct((TB, CODE_DIM), jnp.float32)],
        scratch_shapes=_SCRATCH,
        compiler_params=_PARAMS,
    )(weights, W_enc, codebooks)


def _vq_next(q, r_prev, sel_prev, codebooks):
    return pl.pallas_call(
        functools.partial(_next_body, q),
        grid=(NSTEPS,),
        in_specs=[_tok_spec(CODE_DIM), _tok_spec(CODE_DIM), _cb_spec(q)],
        out_specs=[_tok_spec(1), _tok_spec(1), _tok_spec(CODE_DIM)],
        out_shape=_IDX_OUT + [
            jax.ShapeDtypeStruct((TB, CODE_DIM), jnp.float32)],
        scratch_shapes=_SCRATCH,
        compiler_params=_PARAMS,
    )(r_prev, sel_prev, codebooks)


def _vq_last(q, r_prev, sel_prev, codebooks):
    return pl.pallas_call(
        functools.partial(_last_body, q),
        grid=(NSTEPS,),
        in_specs=[_tok_spec(CODE_DIM), _tok_spec(CODE_DIM), _cb_spec(q)],
        out_specs=[_tok_spec(1)],
        out_shape=[jax.ShapeDtypeStruct((TB, 1), jnp.float32)],
        scratch_shapes=_SCRATCH,
        compiler_params=_PARAMS,
    )(r_prev, sel_prev, codebooks)


_GATHER_CACHE = {}


def _sc_gather(flat_cb, idx):
    n = idx.shape[0]
    if n not in _GATHER_CACHE:
        info = plsc.get_sparse_core_info()
        nw = info.num_cores * info.num_subcores
        b_per_w = n // nw
        mesh = plsc.VectorSubcoreMesh(core_axis_name="c",
                                      subcore_axis_name="s")

        @functools.partial(
            pl.kernel,
            mesh=mesh,
            out_type=jax.ShapeDtypeStruct((n, CODE_DIM), jnp.float32),
            scratch_types=[
                pltpu.VMEM((b_per_w,), jnp.int32),
                pltpu.VMEM((b_per_w, CODE_DIM), jnp.float32),
                pltpu.SemaphoreType.DMA,
            ],
        )
        def gather_kernel(table_hbm, idx_hbm, out_hbm, idx_v, rows_v, sem):
            wid = lax.axis_index("s") * info.num_cores + lax.axis_index("c")
            base = wid * b_per_w
            pltpu.sync_copy(idx_hbm.at[pl.ds(base, b_per_w)], idx_v)
            pltpu.async_copy(table_hbm.at[idx_v], rows_v, sem).wait()
            pltpu.sync_copy(rows_v, out_hbm.at[pl.ds(base, b_per_w)])

        _GATHER_CACHE[n] = gather_kernel
    return _GATHER_CACHE[n](flat_cb, idx)


def kernel(weights, y, W_enc, codebooks):
    flat_cb = codebooks.reshape(NUM_Q * K, CODE_DIM)
    idx_cols = [[] for _ in range(NHALF)]
    r = [None] * NHALF
    idxi = [None] * NHALF
    for h in range(NHALF):
        f, ii, rr = _vq_first(h, weights, W_enc, codebooks)
        idx_cols[h].append(f)
        idxi[h], r[h] = ii, rr
    for q in range(1, NUM_Q):
        for h in range(NHALF):
            sel = _sc_gather(flat_cb, idxi[h].reshape(TB))
            if q < NUM_Q - 1:
                f, ii, rr = _vq_next(q, r[h], sel, codebooks)
                idxi[h], r[h] = ii, rr
            else:
                (f,) = _vq_last(q, r[h], sel, codebooks)
            idx_cols[h].append(f)
    indices = jnp.concatenate(
        [jnp.concatenate(cols, axis=1) for cols in idx_cols], axis=0)
    bos = jnp.array([K], dtype=jnp.float32)
    eos = jnp.array([K + 1], dtype=jnp.float32)
    x = jnp.concatenate([bos, indices.reshape(-1), eos])
    return (x, y)
